# baseline (speedup 1.0000x reference)
import functools

import jax
import jax.numpy as jnp
from jax import lax
from jax.experimental import pallas as pl
from jax.experimental.pallas import tpu as pltpu
from jax.experimental.pallas import tpu_sc as plsc

L = 16
NC = 2
NS = 16
NW = NC * NS
CH = 128
NBUF = 10
TH = 4096


def _combined_tc(type_table, pos_embedding):
    T, D = type_table.shape
    S = pos_embedding.shape[1]

    def body(type_ref, pos_ref, out_ref):
        t = type_ref[...]
        p = pos_ref[...]
        out_ref[...] = t[:, None, :] + p[0][None, :, :]

    out = pl.pallas_call(
        body,
        out_shape=jax.ShapeDtypeStruct((T, S, D), jnp.float32),
    )(type_table, pos_embedding)
    return out.reshape(T * S, D)


def _sc_lookup(tok_flat, tt_flat, pos_flat, word_table, combined, seq_len):
    N = tok_flat.shape[0]
    D = word_table.shape[1]
    S = seq_len
    per_w = N // NW
    n_ch = per_w // CH
    assert per_w % CH == 0

    mesh = plsc.VectorSubcoreMesh(core_axis_name="c", subcore_axis_name="s")
    SKW = 3
    n_outer = (n_ch + 3 * SKW + NBUF - 1) // NBUF

    @functools.partial(
        pl.kernel,
        out_type=jax.ShapeDtypeStruct((N, D), jnp.float32),
        mesh=mesh,
        compiler_params=pltpu.CompilerParams(use_tc_tiling_on_sc=False),
        scratch_types=[
            pltpu.VMEM((NBUF, CH), jnp.int32),
            pltpu.VMEM((NBUF, CH), jnp.int32),
            pltpu.VMEM((NBUF, CH), jnp.int32),
            pltpu.VMEM((NBUF, CH), jnp.int32),
            pltpu.VMEM((NBUF, CH, D), jnp.float32),
        ] + [pltpu.SemaphoreType.DMA] * NBUF,
    )
    def sc_k(tok_hbm, tt_hbm, pos_hbm, word_hbm, comb_hbm, out_hbm,
             tok_v, tt_v, pos_v, cidx_v, rows_v, *sems):
        sem_i = sem_g = sem_a = sem_w = sems
        wid = lax.axis_index("s") * NC + lax.axis_index("c")
        base = wid * per_w

        def valid(cc, fn):
            @pl.when(jnp.logical_and(cc >= 0, cc < n_ch))
            def _():
                fn()

        def outer(g, carry):
            c0 = g * NBUF
            for s in range(NBUF):
                c_fire = c0 + s

                def st0(cc=c_fire, s0=s):
                    off = base + cc * CH
                    pltpu.async_copy(tok_hbm.at[pl.ds(off, CH)], tok_v.at[s0], sem_i[s0])
                    pltpu.async_copy(tt_hbm.at[pl.ds(off, CH)], tt_v.at[s0], sem_i[s0])
                    pltpu.async_copy(pos_hbm.at[pl.ds(off, CH)], pos_v.at[s0], sem_i[s0])

                def st0w(cc=c_fire - NBUF, s0=s):
                    off = base + cc * CH
                    pltpu.make_async_copy(rows_v.at[s0], out_hbm.at[pl.ds(off, CH)], sem_w[s0]).wait()

                def st1(cc=c_fire - SKW, s1=(s - SKW) % NBUF):
                    off = base + cc * CH
                    pltpu.make_async_copy(tok_hbm.at[pl.ds(off, CH)], tok_v.at[s1], sem_i[s1]).wait()
                    pltpu.make_async_copy(tt_hbm.at[pl.ds(off, CH)], tt_v.at[s1], sem_i[s1]).wait()
                    pltpu.make_async_copy(pos_hbm.at[pl.ds(off, CH)], pos_v.at[s1], sem_i[s1]).wait()
                    for k in range(CH // L):
                        sl = pl.ds(k * L, L)
                        v = tok_v[s1, sl]
                        j = v & (2 * TH - 1)
                        tok_v[s1, sl] = (v - j) + 2 * j - jnp.where(j < TH, 0, 2 * TH - 1)
                        cidx_v[s1, sl] = tt_v[s1, sl] * S + pos_v[s1, sl]
                    pltpu.async_copy(word_hbm.at[tok_v.at[s1]], rows_v.at[s1], sem_g[s1])

                def st2(s2=(s - 2 * SKW) % NBUF):
                    pltpu.make_async_copy(word_hbm.at[tok_v.at[s2]], rows_v.at[s2], sem_g[s2]).wait()
                    pltpu.async_copy(comb_hbm.at[cidx_v.at[s2]], rows_v.at[s2], sem_a[s2], add=True)

                def st3(cc=c_fire - 3 * SKW, s3=(s - 3 * SKW) % NBUF):
                    off = base + cc * CH
                    pltpu.make_async_copy(comb_hbm.at[cidx_v.at[s3]], rows_v.at[s3], sem_a[s3]).wait()
                    pltpu.async_copy(rows_v.at[s3], out_hbm.at[pl.ds(off, CH)], sem_w[s3])

                valid(c_fire - NBUF, st0w)
                valid(c_fire, st0)
                valid(c_fire - SKW, st1)
                valid(c_fire - 2 * SKW, st2)
                valid(c_fire - 3 * SKW, st3)
            return carry

        lax.fori_loop(0, n_outer, outer, 0)

    return sc_k(tok_flat, tt_flat, pos_flat, word_table, combined)


def _transpose_table_tc(word_table):
    D, V = word_table.T.shape
    wt_T = word_table.T
    H = TH
    VB = 2 * H
    grid = pl.cdiv(V, VB)

    def body(in_ref, out_ref):
        t = in_ref[...].T
        out_ref[...] = jnp.concatenate([t[:H], t[H:]], axis=1)

    return pl.pallas_call(
        body,
        grid=(grid,),
        in_specs=[pl.BlockSpec((D, VB), lambda i: (0, i))],
        out_specs=pl.BlockSpec((H, 2 * D), lambda i: (i, 0)),
        out_shape=jax.ShapeDtypeStruct((grid * H, 2 * D), jnp.float32),
    )(wt_T)


def kernel(tokens, token_types, word_table, type_table, pos_embedding):
    B, S = tokens.shape
    D = word_table.shape[1]
    V = word_table.shape[0]
    tok_flat = tokens.reshape(-1).astype(jnp.int32)
    tt_flat = token_types.reshape(-1).astype(jnp.int32)
    pos_flat = jnp.broadcast_to(
        jnp.arange(S, dtype=jnp.int32)[None, :], (B, S)).reshape(-1)
    combined = _combined_tc(type_table.astype(jnp.float32),
                            pos_embedding.astype(jnp.float32))
    wt_pairs = _transpose_table_tc(word_table)
    wt_rows = wt_pairs.reshape(wt_pairs.shape[0] * 2, D)
    N = tok_flat.shape[0]
    Nh = N // 2
    halves = [
        _sc_lookup(tok_flat[h * Nh:(h + 1) * Nh], tt_flat[h * Nh:(h + 1) * Nh],
                   pos_flat[h * Nh:(h + 1) * Nh], wt_rows, combined, S)
        for h in range(2)
    ]
    out = jnp.concatenate(halves, axis=0)
    return out.reshape(B, S, D)


# speedup vs baseline: 1.1105x; 1.1105x over previous
import functools

import jax
import jax.numpy as jnp
from jax import lax
from jax.experimental import pallas as pl
from jax.experimental.pallas import tpu as pltpu
from jax.experimental.pallas import tpu_sc as plsc

L = 16
NC = 2
NS = 16
NW = NC * NS
CH = 128
NBUF = 10
TH = 4096


def _combined_tc(type_table, pos_embedding):
    T, D = type_table.shape
    S = pos_embedding.shape[1]

    def body(type_ref, pos_ref, out_ref):
        t = type_ref[...]
        p = pos_ref[...]
        out_ref[...] = t[:, None, :] + p[0][None, :, :]

    out = pl.pallas_call(
        body,
        out_shape=jax.ShapeDtypeStruct((T, S, D), jnp.float32),
    )(type_table, pos_embedding)
    return out.reshape(T * S, D)


def _sc_lookup(tok_flat, tt_flat, pos_flat, word_table, combined, seq_len):
    N = tok_flat.shape[0]
    D = word_table.shape[1]
    S = seq_len
    per_w = N // NW
    n_ch = per_w // CH
    assert per_w % CH == 0

    mesh = plsc.VectorSubcoreMesh(core_axis_name="c", subcore_axis_name="s")
    SKW = 3
    n_outer = (n_ch + 3 * SKW + NBUF - 1) // NBUF

    @functools.partial(
        pl.kernel,
        out_type=jax.ShapeDtypeStruct((N, D), jnp.float32),
        mesh=mesh,
        compiler_params=pltpu.CompilerParams(use_tc_tiling_on_sc=False),
        scratch_types=[
            pltpu.VMEM((NBUF, CH), jnp.int32),
            pltpu.VMEM((NBUF, CH), jnp.int32),
            pltpu.VMEM((NBUF, CH), jnp.int32),
            pltpu.VMEM((NBUF, CH), jnp.int32),
            pltpu.VMEM((NBUF, CH, D), jnp.float32),
        ] + [pltpu.SemaphoreType.DMA] * NBUF,
    )
    def sc_k(tok_hbm, tt_hbm, pos_hbm, word_hbm, comb_hbm, out_hbm,
             tok_v, tt_v, pos_v, cidx_v, rows_v, *sems):
        sem_i = sem_g = sem_a = sem_w = sems
        wid = lax.axis_index("s") * NC + lax.axis_index("c")
        base = wid * per_w

        def valid(cc, fn):
            @pl.when(jnp.logical_and(cc >= 0, cc < n_ch))
            def _():
                fn()

        def outer(g, carry):
            c0 = g * NBUF
            for s in range(NBUF):
                c_fire = c0 + s

                def st0(cc=c_fire, s0=s):
                    off = base + cc * CH
                    pltpu.async_copy(tok_hbm.at[pl.ds(off, CH)], tok_v.at[s0], sem_i[s0])
                    pltpu.async_copy(tt_hbm.at[pl.ds(off, CH)], tt_v.at[s0], sem_i[s0])
                    pltpu.async_copy(pos_hbm.at[pl.ds(off, CH)], pos_v.at[s0], sem_i[s0])

                def st0w(cc=c_fire - NBUF, s0=s):
                    off = base + cc * CH
                    pltpu.make_async_copy(rows_v.at[s0], out_hbm.at[pl.ds(off, CH)], sem_w[s0]).wait()

                def st1(cc=c_fire - SKW, s1=(s - SKW) % NBUF):
                    off = base + cc * CH
                    pltpu.make_async_copy(tok_hbm.at[pl.ds(off, CH)], tok_v.at[s1], sem_i[s1]).wait()
                    pltpu.make_async_copy(tt_hbm.at[pl.ds(off, CH)], tt_v.at[s1], sem_i[s1]).wait()
                    pltpu.make_async_copy(pos_hbm.at[pl.ds(off, CH)], pos_v.at[s1], sem_i[s1]).wait()
                    for k in range(CH // L):
                        sl = pl.ds(k * L, L)
                        v = tok_v[s1, sl]
                        j = v & (2 * TH - 1)
                        tok_v[s1, sl] = (v - j) + 2 * j - jnp.where(j < TH, 0, 2 * TH - 1)
                        cidx_v[s1, sl] = tt_v[s1, sl] * S + pos_v[s1, sl]
                    pltpu.async_copy(word_hbm.at[tok_v.at[s1]], rows_v.at[s1], sem_g[s1])

                def st2(s2=(s - 2 * SKW) % NBUF):
                    pltpu.make_async_copy(word_hbm.at[tok_v.at[s2]], rows_v.at[s2], sem_g[s2]).wait()
                    pltpu.async_copy(comb_hbm.at[cidx_v.at[s2]], rows_v.at[s2], sem_a[s2], add=True)

                def st3(cc=c_fire - 3 * SKW, s3=(s - 3 * SKW) % NBUF):
                    off = base + cc * CH
                    pltpu.make_async_copy(comb_hbm.at[cidx_v.at[s3]], rows_v.at[s3], sem_a[s3]).wait()
                    pltpu.async_copy(rows_v.at[s3], out_hbm.at[pl.ds(off, CH)], sem_w[s3])

                valid(c_fire - NBUF, st0w)
                valid(c_fire, st0)
                valid(c_fire - SKW, st1)
                valid(c_fire - 2 * SKW, st2)
                valid(c_fire - 3 * SKW, st3)
            return carry

        lax.fori_loop(0, n_outer, outer, 0)

    return sc_k(tok_flat, tt_flat, pos_flat, word_table, combined)


def _transpose_table_tc(word_table):
    D, V = word_table.T.shape
    wt_T = word_table.T
    H = TH
    VB = 2 * H
    grid = pl.cdiv(V, VB)

    def body(in_ref, out_ref):
        t = in_ref[...].T
        out_ref[...] = jnp.concatenate([t[:H], t[H:]], axis=1)

    return pl.pallas_call(
        body,
        grid=(grid,),
        in_specs=[pl.BlockSpec((D, VB), lambda i: (0, i))],
        out_specs=pl.BlockSpec((H, 2 * D), lambda i: (i, 0)),
        out_shape=jax.ShapeDtypeStruct((grid * H, 2 * D), jnp.float32),
    )(wt_T)


def kernel(tokens, token_types, word_table, type_table, pos_embedding):
    B, S = tokens.shape
    D = word_table.shape[1]
    V = word_table.shape[0]
    tok_flat = tokens.reshape(-1).astype(jnp.int32)
    tt_flat = token_types.reshape(-1).astype(jnp.int32)
    pos_flat = jnp.broadcast_to(
        jnp.arange(S, dtype=jnp.int32)[None, :], (B, S)).reshape(-1)
    combined = _combined_tc(type_table.astype(jnp.float32),
                            pos_embedding.astype(jnp.float32))
    wt_pairs = _transpose_table_tc(word_table)
    wt_rows = wt_pairs.reshape(wt_pairs.shape[0] * 2, D)
    out = _sc_lookup(tok_flat, tt_flat, pos_flat, wt_rows, combined, S)
    return out.reshape(B, S, D)
